# baseline (speedup 1.0000x reference)
import jax
import jax.numpy as jnp
from jax import lax
from jax.experimental import pallas as pl

_K = 64
_BLOCK_B = 8


def _body(x_ref, o_ref):
    xv = x_ref[...]
    i = lax.bitcast_convert_type(xv, jnp.int32)
    key = i ^ (lax.shift_right_arithmetic(i, 31) & jnp.int32(0x7FFFFFFF))

    sign = jnp.int32(-2147483648)

    def cond(state):
        t, _, cur = state
        return jnp.logical_and(t < 32, jnp.any(cur != _K))

    def step(state):
        t, uprefix, cur = state
        bit = jnp.int32(1) << (jnp.int32(31) - t)
        ucand = uprefix | bit
        m = (key >= (ucand ^ sign)).astype(jnp.int32)
        n = m.shape[1]
        while n > 128:
            m = m.reshape(m.shape[0], 2, n // 2).sum(axis=1)
            n //= 2
        cnt = jnp.sum(m, axis=1, keepdims=True)
        take = cnt >= _K
        return (t + 1,
                jnp.where(take, ucand, uprefix),
                jnp.where(take, cnt, cur))

    BB = xv.shape[0]
    init = (jnp.int32(0),
            jnp.zeros((BB, 1), jnp.int32),
            jnp.full((BB, 1), jnp.int32(xv.shape[1])))
    _, uthresh, _ = lax.while_loop(cond, step, init)
    thresh = uthresh ^ sign

    o_ref[...] = jnp.where(key >= thresh, jnp.maximum(xv, 0.0), 0.0)


def kernel(x):
    B, N = x.shape
    grid = (B // _BLOCK_B,)
    return pl.pallas_call(
        _body,
        grid=grid,
        in_specs=[pl.BlockSpec((_BLOCK_B, N), lambda b: (b, 0))],
        out_specs=pl.BlockSpec((_BLOCK_B, N), lambda b: (b, 0)),
        out_shape=jax.ShapeDtypeStruct((B, N), x.dtype),
    )(x)


# speedup vs baseline: 11.8905x; 11.8905x over previous
import jax
import jax.numpy as jnp
from jax import lax
from jax.experimental import pallas as pl

_K = 64
_BLOCK_B = 8


def _body(x_ref, o_ref):
    xv = x_ref[...]
    i = lax.bitcast_convert_type(xv, jnp.int32)
    key = i ^ (lax.shift_right_arithmetic(i, 31) & jnp.int32(0x7FFFFFFF))

    sign = jnp.int32(-2147483648)

    def cond(state):
        t, _, cur = state
        return jnp.logical_and(t < 16, jnp.any(cur != _K))

    def count(cand):
        return jnp.sum((key >= (cand ^ sign)).astype(jnp.int32),
                       axis=1, keepdims=True)

    def step(state):
        t, uprefix, cur = state
        b1 = jnp.int32(1) << (jnp.int32(31) - 2 * t)
        b0 = jnp.int32(1) << (jnp.int32(30) - 2 * t)
        c3 = uprefix | b1 | b0
        c2 = uprefix | b1
        c1 = uprefix | b0
        n3, n2, n1 = count(c3), count(c2), count(c1)
        take3 = n3 >= _K
        take2 = jnp.logical_and(~take3, n2 >= _K)
        take1 = jnp.logical_and(~(take3 | take2), n1 >= _K)
        newp = jnp.where(take3, c3,
                         jnp.where(take2, c2, jnp.where(take1, c1, uprefix)))
        newc = jnp.where(take3, n3,
                         jnp.where(take2, n2, jnp.where(take1, n1, cur)))
        return (t + 1, newp, newc)

    BB = xv.shape[0]
    init = (jnp.int32(0),
            jnp.zeros((BB, 1), jnp.int32),
            jnp.full((BB, 1), jnp.int32(xv.shape[1])))
    _, uthresh, _ = lax.while_loop(cond, step, init)
    thresh = uthresh ^ sign

    o_ref[...] = jnp.where(key >= thresh, jnp.maximum(xv, 0.0), 0.0)


def kernel(x):
    B, N = x.shape
    grid = (B // _BLOCK_B,)
    return pl.pallas_call(
        _body,
        grid=grid,
        in_specs=[pl.BlockSpec((_BLOCK_B, N), lambda b: (b, 0))],
        out_specs=pl.BlockSpec((_BLOCK_B, N), lambda b: (b, 0)),
        out_shape=jax.ShapeDtypeStruct((B, N), x.dtype),
    )(x)
